# Initial kernel scaffold; baseline (speedup 1.0000x reference)
import jax
import jax.numpy as jnp
from jax.experimental import pallas as pl


def kernel(node_embeddings, rel2_indices, rel1_indices, rel2_W1, rel2_b1, rel2_W2, rel2_b2, rel1_W1, rel1_b1, rel1_W2, rel1_b2, upd_W1, upd_b1, upd_W2, upd_b2, ln_g, ln_b):
    raise NotImplementedError("write your pallas kernel here")


# speedup vs baseline: 1.5617x; 1.5617x over previous
import jax
import jax.numpy as jnp
from jax.experimental import pallas as pl

_N = 10000
_D = 128
_LAYERS = 3
_NB = 1000
_EB = 2000


def _prologue_body(h_ref, w1a_ref, w1b_ref, b1_ref, r1w1_ref, r1b1_ref,
                   r1w2_ref, r1b2_ref, a_ref, b_ref, g_ref):
    h = h_ref[...]
    a_ref[...] = jnp.dot(h, w1a_ref[...], preferred_element_type=jnp.float32) + b1_ref[...]
    b_ref[...] = jnp.dot(h, w1b_ref[...], preferred_element_type=jnp.float32)
    t = jax.nn.relu(jnp.dot(h, r1w1_ref[...], preferred_element_type=jnp.float32) + r1b1_ref[...])
    g_ref[...] = jnp.dot(t, r1w2_ref[...], preferred_element_type=jnp.float32) + r1b2_ref[...]


def _edge_body(z_ref, w2_ref, b2_ref, f_ref):
    f_ref[...] = (jnp.dot(jax.nn.relu(z_ref[...]), w2_ref[...],
                          preferred_element_type=jnp.float32) + b2_ref[...])


def _update_body(m_ref, g_ref, mask_ref, h_ref, wua_ref, wub_ref, ub1_ref,
                 uw2_ref, ub2_ref, lng_ref, lnb_ref, o_ref):
    m = m_ref[...]
    g = g_ref[...]
    mask = mask_ref[...]
    m = jnp.maximum(m, jnp.where(mask > 0.0, g, -jnp.inf))
    h = h_ref[...]
    x = h + m
    t = jax.nn.relu(jnp.dot(x, wua_ref[...], preferred_element_type=jnp.float32)
                    + jnp.dot(h, wub_ref[...], preferred_element_type=jnp.float32)
                    + ub1_ref[...])
    u = jnp.dot(t, uw2_ref[...], preferred_element_type=jnp.float32) + ub2_ref[...]
    mu = jnp.mean(u, axis=-1, keepdims=True)
    var = jnp.mean((u - mu) ** 2, axis=-1, keepdims=True)
    u = (u - mu) * jax.lax.rsqrt(var + 1e-5) * lng_ref[...] + lnb_ref[...]
    o_ref[...] = h + u


def _full(block):
    return pl.BlockSpec(block, lambda i: (0, 0))


def _rows(block):
    return pl.BlockSpec(block, lambda i: (i, 0))


def _prologue(h, w1a, w1b, b1, r1w1, r1b1, r1w2, r1b2):
    return pl.pallas_call(
        _prologue_body,
        grid=(_N // _NB,),
        in_specs=[_rows((_NB, _D)), _full((_D, 2 * _D)), _full((_D, 2 * _D)),
                  _full((1, 2 * _D)), _full((_D, _D)), _full((1, _D)),
                  _full((_D, _D)), _full((1, _D))],
        out_specs=[_rows((_NB, 2 * _D)), _rows((_NB, 2 * _D)), _rows((_NB, _D))],
        out_shape=[jax.ShapeDtypeStruct((_N, 2 * _D), jnp.float32),
                   jax.ShapeDtypeStruct((_N, 2 * _D), jnp.float32),
                   jax.ShapeDtypeStruct((_N, _D), jnp.float32)],
    )(h, w1a, w1b, b1, r1w1, r1b1, r1w2, r1b2)


def _edge_mlp(z, w2, b2):
    e = z.shape[0]
    return pl.pallas_call(
        _edge_body,
        grid=(e // _EB,),
        in_specs=[_rows((_EB, 2 * _D)), _full((2 * _D, 2 * _D)), _full((1, 2 * _D))],
        out_specs=_rows((_EB, 2 * _D)),
        out_shape=jax.ShapeDtypeStruct((e, 2 * _D), jnp.float32),
    )(z, w2, b2)


def _update(m, g, mask, h, wua, wub, ub1, uw2, ub2, lng, lnb):
    return pl.pallas_call(
        _update_body,
        grid=(_N // _NB,),
        in_specs=[_rows((_NB, _D)), _rows((_NB, _D)), _rows((_NB, 1)),
                  _rows((_NB, _D)), _full((_D, 2 * _D)), _full((_D, 2 * _D)),
                  _full((1, 2 * _D)), _full((2 * _D, _D)), _full((1, _D)),
                  _full((1, _D)), _full((1, _D))],
        out_specs=_rows((_NB, _D)),
        out_shape=jax.ShapeDtypeStruct((_N, _D), jnp.float32),
    )(m, g, mask, h, wua, wub, ub1, uw2, ub2, lng, lnb)


def kernel(node_embeddings, rel2_indices, rel1_indices, rel2_W1, rel2_b1,
           rel2_W2, rel2_b2, rel1_W1, rel1_b1, rel1_W2, rel1_b2, upd_W1,
           upd_b1, upd_W2, upd_b2, ln_g, ln_b):
    h = node_embeddings
    w1a, w1b = rel2_W1[:_D], rel2_W1[_D:]
    wua, wub = upd_W1[:_D], upd_W1[_D:]
    b1 = rel2_b1.reshape(1, -1)
    b2 = rel2_b2.reshape(1, -1)
    r1b1 = rel1_b1.reshape(1, -1)
    r1b2 = rel1_b2.reshape(1, -1)
    ub1 = upd_b1.reshape(1, -1)
    ub2 = upd_b2.reshape(1, -1)
    lng = ln_g.reshape(1, -1)
    lnb = ln_b.reshape(1, -1)
    src = rel2_indices[0::2]
    dst = rel2_indices[1::2]
    mask = jnp.zeros((_N, 1), jnp.float32).at[rel1_indices].set(1.0)
    for _ in range(_LAYERS):
        a, b, g = _prologue(h, w1a, w1b, b1, rel1_W1, r1b1, rel1_W2, r1b2)
        z = jnp.take(a, src, axis=0) + jnp.take(b, dst, axis=0)
        f = _edge_mlp(z, rel2_W2, b2)
        m = jnp.full((_N, _D), -jnp.inf, jnp.float32).at[rel2_indices].max(
            f.reshape(-1, _D))
        h = _update(m, g, mask, h, wua, wub, ub1, uw2=upd_W2, ub2=ub2,
                    lng=lng, lnb=lnb)
    return h
